# baseline (speedup 1.0000x reference)
import jax
import jax.numpy as jnp
from jax import lax
from jax.experimental import pallas as pl
from jax.experimental.pallas import tpu as pltpu
from jax.experimental.pallas import tpu_sc as plsc

N_NODES = 10000
N_EDGES = 320000
D = 128
NC = 2
NS = 16
NW = NC * NS
CH = 128
EPW = 10240
K = EPW // CH
E_PAD = NW * EPW
NP = 10112
ROWS_PT = NP // NS

EB = 2500
HG = N_EDGES // EB

_mesh = plsc.VectorSubcoreMesh(core_axis_name="c", subcore_axis_name="s")


def _seg_sum_body(table, srcs, dsts, zrows, sums_out,
                  acc, src_v, dst_v, rows_v, sem_g0, sem_g1):
    cid = lax.axis_index("c")
    sid = lax.axis_index("s")
    wid = sid * NC + cid
    r0 = sid * ROWS_PT
    sems = (sem_g0, sem_g1)

    pltpu.sync_copy(zrows.at[pl.ds(r0, ROWS_PT)], acc.at[pl.ds(r0, ROWS_PT)])
    plsc.subcore_barrier()

    def fetch_idx(n, b):
        pltpu.sync_copy(srcs.at[n], src_v.at[b])
        pltpu.sync_copy(dsts.at[n], dst_v.at[b])

    def start_gather(b):
        pltpu.async_copy(table.at[src_v.at[b]], rows_v.at[b], sems[b])

    def wait_gather(b):
        pltpu.make_async_copy(table.at[src_v.at[b]], rows_v.at[b],
                              sems[b]).wait()

    fetch_idx(wid * K, 0)
    start_gather(0)

    @pl.loop(0, K, step=2)
    def _(j):
        n = wid * K + j
        fetch_idx(n + 1, 1)
        start_gather(1)
        wait_gather(0)
        pltpu.sync_copy(rows_v.at[0], acc.at[dst_v.at[0]], add=True)

        @pl.when(j + 2 < K)
        def _():
            fetch_idx(n + 2, 0)
            start_gather(0)

        wait_gather(1)
        pltpu.sync_copy(rows_v.at[1], acc.at[dst_v.at[1]], add=True)

    plsc.subcore_barrier()
    pltpu.sync_copy(acc.at[pl.ds(r0, ROWS_PT)],
                    sums_out.at[cid, pl.ds(r0, ROWS_PT)])


_seg_sum = pl.kernel(
    _seg_sum_body,
    out_type=[jax.ShapeDtypeStruct((NC, NP, D), jnp.float32)],
    mesh=_mesh,
    scratch_types=[
        pltpu.VMEM_SHARED((NP, D), jnp.float32),
        pltpu.VMEM((2, CH), jnp.int32),
        pltpu.VMEM((2, CH), jnp.int32),
        pltpu.VMEM((2, CH, D), jnp.float32),
        pltpu.SemaphoreType.DMA,
        pltpu.SemaphoreType.DMA,
    ],
)


def _hist_body(dst_ref, out_ref):
    row = dst_ref[0]
    a = row // 128
    b = row - a * 128
    ii = lax.broadcasted_iota(jnp.int32, (128, EB), 0)
    a_oh = jnp.where(a == ii, 1.0, 0.0).astype(jnp.bfloat16)
    b_oh = jnp.where(b == ii, 1.0, 0.0).astype(jnp.bfloat16)
    part = lax.dot_general(a_oh, b_oh, (((1,), (1,)), ((), ())),
                           preferred_element_type=jnp.float32)

    @pl.when(pl.program_id(0) == 0)
    def _():
        out_ref[...] = jnp.zeros_like(out_ref)

    out_ref[...] += part


_hist = pl.pallas_call(
    _hist_body,
    grid=(HG,),
    in_specs=[pl.BlockSpec((1, 1, EB), lambda i: (i, 0, 0))],
    out_specs=pl.BlockSpec((128, 128), lambda i: (0, 0)),
    out_shape=jax.ShapeDtypeStruct((128, 128), jnp.float32),
)


def _tc1_body(sums, cnt, x, wl1, wr1, b1, wl2, wr2, b2, hw_out, hr_out):
    c = jnp.maximum(cnt[...], 1.0)
    mean = (sums[0, :N_NODES, :] + sums[1, :N_NODES, :]) / c
    h = mean @ wl1[...] + x[...] @ wr1[...] + b1[...][None, :]
    h = jnp.maximum(h, 0.0)
    hw_out[...] = h @ wl2[...]
    hr_out[...] = h @ wr2[...] + b2[...][None, :]


def _tc2_body(sums, cnt, hr, out):
    c = jnp.maximum(cnt[...], 1.0)
    out[...] = (sums[0, :N_NODES, :] + sums[1, :N_NODES, :]) / c + hr[...]


_tc1 = pl.pallas_call(
    _tc1_body,
    out_shape=(
        jax.ShapeDtypeStruct((N_NODES, D), jnp.float32),
        jax.ShapeDtypeStruct((N_NODES, D), jnp.float32),
    ),
)

_tc2 = pl.pallas_call(
    _tc2_body,
    out_shape=jax.ShapeDtypeStruct((N_NODES, D), jnp.float32),
)


def kernel(x, edge_index, Wl1, Wr1, b1, Wl2, Wr2, b2):
    src = edge_index[0].astype(jnp.int32)
    dst = edge_index[1].astype(jnp.int32)
    pad = E_PAD - N_EDGES
    srcs = jnp.concatenate(
        [src, jnp.zeros((pad,), jnp.int32)]).reshape(NW * K, CH)
    dsts = jnp.concatenate(
        [dst, jnp.full((pad,), N_NODES, jnp.int32)]).reshape(NW * K, CH)

    zrows = jnp.zeros((NP, D), jnp.float32)

    cnt_grid = _hist(dst.reshape(HG, 1, EB))
    cnt = cnt_grid.reshape(-1)[:N_NODES, None]

    (sums1,) = _seg_sum(x, srcs, dsts, zrows)
    hw, hr = _tc1(sums1, cnt, x, Wl1, Wr1, b1, Wl2, Wr2, b2)
    (sums2,) = _seg_sum(hw, srcs, dsts, zrows)
    return _tc2(sums2, cnt, hr)


# speedup vs baseline: 1.0009x; 1.0009x over previous
import jax
import jax.numpy as jnp
from jax import lax
from jax.experimental import pallas as pl
from jax.experimental.pallas import tpu as pltpu
from jax.experimental.pallas import tpu_sc as plsc

N_NODES = 10000
N_EDGES = 320000
D = 128
NC = 2
NS = 16
NW = NC * NS
CH = 128
EPW = 10240
K = EPW // CH
E_PAD = NW * EPW
NP = 10112
ROWS_PT = NP // NS

EB = 2500
HG = N_EDGES // EB

_mesh = plsc.VectorSubcoreMesh(core_axis_name="c", subcore_axis_name="s")


def _seg_sum_body(table, srcs, dsts, zrows, sums_out,
                  acc, src_v0, src_v1, dst_v0, dst_v1, rows_v0, rows_v1,
                  sem_g0, sem_g1):
    cid = lax.axis_index("c")
    sid = lax.axis_index("s")
    wid = sid * NC + cid
    r0 = sid * ROWS_PT
    src_b = (src_v0, src_v1)
    dst_b = (dst_v0, dst_v1)
    rows_b = (rows_v0, rows_v1)
    sems = (sem_g0, sem_g1)

    pltpu.sync_copy(zrows.at[pl.ds(r0, ROWS_PT)], acc.at[pl.ds(r0, ROWS_PT)])
    plsc.subcore_barrier()

    def fetch_idx(n, b):
        pltpu.sync_copy(srcs.at[n], src_b[b])
        pltpu.sync_copy(dsts.at[n], dst_b[b])

    def start_gather(b):
        pltpu.async_copy(table.at[src_b[b]], rows_b[b], sems[b])

    def wait_gather(b):
        pltpu.make_async_copy(table.at[src_b[b]], rows_b[b], sems[b]).wait()

    fetch_idx(wid * K, 0)
    start_gather(0)

    @pl.loop(0, K, step=2)
    def _(j):
        n = wid * K + j
        fetch_idx(n + 1, 1)
        start_gather(1)
        wait_gather(0)
        pltpu.sync_copy(rows_b[0], acc.at[dst_b[0]], add=True)

        @pl.when(j + 2 < K)
        def _():
            fetch_idx(n + 2, 0)
            start_gather(0)

        wait_gather(1)
        pltpu.sync_copy(rows_b[1], acc.at[dst_b[1]], add=True)

    plsc.subcore_barrier()
    pltpu.sync_copy(acc.at[pl.ds(r0, ROWS_PT)],
                    sums_out.at[cid, pl.ds(r0, ROWS_PT)])


_seg_sum = pl.kernel(
    _seg_sum_body,
    out_type=[jax.ShapeDtypeStruct((NC, NP, D), jnp.float32)],
    mesh=_mesh,
    scratch_types=[
        pltpu.VMEM_SHARED((NP, D), jnp.float32),
        pltpu.VMEM((CH,), jnp.int32),
        pltpu.VMEM((CH,), jnp.int32),
        pltpu.VMEM((CH,), jnp.int32),
        pltpu.VMEM((CH,), jnp.int32),
        pltpu.VMEM((CH, D), jnp.float32),
        pltpu.VMEM((CH, D), jnp.float32),
        pltpu.SemaphoreType.DMA,
        pltpu.SemaphoreType.DMA,
    ],
)


def _hist_body(dst_ref, out_ref):
    row = dst_ref[0]
    a = row // 128
    b = row - a * 128
    ii = lax.broadcasted_iota(jnp.int32, (128, EB), 0)
    a_oh = jnp.where(a == ii, 1.0, 0.0).astype(jnp.bfloat16)
    b_oh = jnp.where(b == ii, 1.0, 0.0).astype(jnp.bfloat16)
    part = lax.dot_general(a_oh, b_oh, (((1,), (1,)), ((), ())),
                           preferred_element_type=jnp.float32)

    @pl.when(pl.program_id(0) == 0)
    def _():
        out_ref[...] = jnp.zeros_like(out_ref)

    out_ref[...] += part


_hist = pl.pallas_call(
    _hist_body,
    grid=(HG,),
    in_specs=[pl.BlockSpec((1, 1, EB), lambda i: (i, 0, 0))],
    out_specs=pl.BlockSpec((128, 128), lambda i: (0, 0)),
    out_shape=jax.ShapeDtypeStruct((128, 128), jnp.float32),
)


def _tc1_body(sums, cnt, x, wl1, wr1, b1, wl2, wr2, b2, hw_out, hr_out):
    c = jnp.maximum(cnt[...], 1.0)
    mean = (sums[0, :N_NODES, :] + sums[1, :N_NODES, :]) / c
    h = mean @ wl1[...] + x[...] @ wr1[...] + b1[...][None, :]
    h = jnp.maximum(h, 0.0)
    hw_out[...] = h @ wl2[...]
    hr_out[...] = h @ wr2[...] + b2[...][None, :]


def _tc2_body(sums, cnt, hr, out):
    c = jnp.maximum(cnt[...], 1.0)
    out[...] = (sums[0, :N_NODES, :] + sums[1, :N_NODES, :]) / c + hr[...]


_tc1 = pl.pallas_call(
    _tc1_body,
    out_shape=(
        jax.ShapeDtypeStruct((N_NODES, D), jnp.float32),
        jax.ShapeDtypeStruct((N_NODES, D), jnp.float32),
    ),
)

_tc2 = pl.pallas_call(
    _tc2_body,
    out_shape=jax.ShapeDtypeStruct((N_NODES, D), jnp.float32),
)


def kernel(x, edge_index, Wl1, Wr1, b1, Wl2, Wr2, b2):
    src = edge_index[0].astype(jnp.int32)
    dst = edge_index[1].astype(jnp.int32)
    pad = E_PAD - N_EDGES
    srcs = jnp.concatenate(
        [src, jnp.zeros((pad,), jnp.int32)]).reshape(NW * K, CH)
    dsts = jnp.concatenate(
        [dst, jnp.full((pad,), N_NODES, jnp.int32)]).reshape(NW * K, CH)

    zrows = jnp.zeros((NP, D), jnp.float32)

    cnt_grid = _hist(dst.reshape(HG, 1, EB))
    cnt = cnt_grid.reshape(-1)[:N_NODES, None]

    (sums1,) = _seg_sum(x, srcs, dsts, zrows)
    hw, hr = _tc1(sums1, cnt, x, Wl1, Wr1, b1, Wl2, Wr2, b2)
    (sums2,) = _seg_sum(hw, srcs, dsts, zrows)
    return _tc2(sums2, cnt, hr)


# speedup vs baseline: 1.0169x; 1.0160x over previous
import jax
import jax.numpy as jnp
from jax import lax
from jax.experimental import pallas as pl
from jax.experimental.pallas import tpu as pltpu
from jax.experimental.pallas import tpu_sc as plsc

N_NODES = 10000
N_EDGES = 320000
D = 128
NC = 2
NS = 16
NW = NC * NS
CH = 128
EPW = 10240
K = EPW // CH
E_PAD = NW * EPW
NP = 10112
ROWS_PT = NP // NS

EB = 2500
HG = N_EDGES // EB

_mesh = plsc.VectorSubcoreMesh(core_axis_name="c", subcore_axis_name="s")


NH = 2
K2 = K // NH


def _seg_sum_body(table, srcs, dsts, zrows, sums_out,
                  acc, src_sl, dst_sl, rows0, rows1, sem0, sem1):
    cid = lax.axis_index("c")
    sid = lax.axis_index("s")
    wid = sid * NC + cid
    r0 = sid * ROWS_PT

    pltpu.sync_copy(zrows.at[pl.ds(r0, ROWS_PT)], acc.at[pl.ds(r0, ROWS_PT)])
    plsc.subcore_barrier()

    @pl.loop(0, NH)
    def _(h):
        n = wid * NH + h
        pltpu.sync_copy(srcs.at[n], src_sl)
        pltpu.sync_copy(dsts.at[n], dst_sl)
        pltpu.async_copy(table.at[src_sl.at[0]], rows0, sem0)

        @pl.loop(0, K2, step=2)
        def _(j):
            pltpu.async_copy(table.at[src_sl.at[j + 1]], rows1, sem1)
            pltpu.make_async_copy(table.at[src_sl.at[j]], rows0, sem0).wait()
            pltpu.sync_copy(rows0, acc.at[dst_sl.at[j]], add=True)

            @pl.when(j + 2 < K2)
            def _():
                pltpu.async_copy(table.at[src_sl.at[j + 2]], rows0, sem0)

            pltpu.make_async_copy(table.at[src_sl.at[j + 1]], rows1,
                                  sem1).wait()
            pltpu.sync_copy(rows1, acc.at[dst_sl.at[j + 1]], add=True)

    plsc.subcore_barrier()
    pltpu.sync_copy(acc.at[pl.ds(r0, ROWS_PT)],
                    sums_out.at[cid, pl.ds(r0, ROWS_PT)])


_seg_sum = pl.kernel(
    _seg_sum_body,
    out_type=[jax.ShapeDtypeStruct((NC, NP, D), jnp.float32)],
    mesh=_mesh,
    scratch_types=[
        pltpu.VMEM_SHARED((NP, D), jnp.float32),
        pltpu.VMEM((K2, CH), jnp.int32),
        pltpu.VMEM((K2, CH), jnp.int32),
        pltpu.VMEM((CH, D), jnp.float32),
        pltpu.VMEM((CH, D), jnp.float32),
        pltpu.SemaphoreType.DMA,
        pltpu.SemaphoreType.DMA,
    ],
)


def _hist_body(dst_ref, out_ref):
    row = dst_ref[0]
    a = row // 128
    b = row - a * 128
    ii = lax.broadcasted_iota(jnp.int32, (128, EB), 0)
    a_oh = jnp.where(a == ii, 1.0, 0.0).astype(jnp.bfloat16)
    b_oh = jnp.where(b == ii, 1.0, 0.0).astype(jnp.bfloat16)
    part = lax.dot_general(a_oh, b_oh, (((1,), (1,)), ((), ())),
                           preferred_element_type=jnp.float32)

    @pl.when(pl.program_id(0) == 0)
    def _():
        out_ref[...] = jnp.zeros_like(out_ref)

    out_ref[...] += part


_hist = pl.pallas_call(
    _hist_body,
    grid=(HG,),
    in_specs=[pl.BlockSpec((1, 1, EB), lambda i: (i, 0, 0))],
    out_specs=pl.BlockSpec((128, 128), lambda i: (0, 0)),
    out_shape=jax.ShapeDtypeStruct((128, 128), jnp.float32),
)


def _tc1_body(sums, cnt, x, wl1, wr1, b1, wl2, wr2, b2, hw_out, hr_out):
    c = jnp.maximum(cnt[...], 1.0)
    mean = (sums[0, :N_NODES, :] + sums[1, :N_NODES, :]) / c
    h = mean @ wl1[...] + x[...] @ wr1[...] + b1[...][None, :]
    h = jnp.maximum(h, 0.0)
    hw_out[...] = h @ wl2[...]
    hr_out[...] = h @ wr2[...] + b2[...][None, :]


def _tc2_body(sums, cnt, hr, out):
    c = jnp.maximum(cnt[...], 1.0)
    out[...] = (sums[0, :N_NODES, :] + sums[1, :N_NODES, :]) / c + hr[...]


_tc1 = pl.pallas_call(
    _tc1_body,
    out_shape=(
        jax.ShapeDtypeStruct((N_NODES, D), jnp.float32),
        jax.ShapeDtypeStruct((N_NODES, D), jnp.float32),
    ),
)

_tc2 = pl.pallas_call(
    _tc2_body,
    out_shape=jax.ShapeDtypeStruct((N_NODES, D), jnp.float32),
)


def kernel(x, edge_index, Wl1, Wr1, b1, Wl2, Wr2, b2):
    src = edge_index[0].astype(jnp.int32)
    dst = edge_index[1].astype(jnp.int32)
    pad = E_PAD - N_EDGES
    srcs = jnp.concatenate(
        [src, jnp.zeros((pad,), jnp.int32)]).reshape(NW * NH, K2, CH)
    dsts = jnp.concatenate(
        [dst, jnp.full((pad,), N_NODES, jnp.int32)]).reshape(NW * NH, K2, CH)

    zrows = jnp.zeros((NP, D), jnp.float32)

    cnt_grid = _hist(dst.reshape(HG, 1, EB))
    cnt = cnt_grid.reshape(-1)[:N_NODES, None]

    (sums1,) = _seg_sum(x, srcs, dsts, zrows)
    hw, hr = _tc1(sums1, cnt, x, Wl1, Wr1, b1, Wl2, Wr2, b2)
    (sums2,) = _seg_sum(hw, srcs, dsts, zrows)
    return _tc2(sums2, cnt, hr)


# speedup vs baseline: 3.6187x; 3.5584x over previous
import jax
import jax.numpy as jnp
from jax import lax
from jax.experimental import pallas as pl
from jax.experimental.pallas import tpu as pltpu
from jax.experimental.pallas import tpu_sc as plsc

N_NODES = 10000
N_EDGES = 320000
D = 128
NC = 2
NS = 16
NW = NC * NS
CH = 128
EPW = 10240
K = EPW // CH
E_PAD = NW * EPW
NP = 10112
ROWS_PT = NP // NS

EB = 2500
HG = N_EDGES // EB

_mesh = plsc.VectorSubcoreMesh(core_axis_name="c", subcore_axis_name="s")


NH = 2
K2 = K // NH


def _seg_sum_body(table, srcs, dsts, zrows, sums_out,
                  acc, src_sl, dst_sl, rows0, rows1, sem0, sem1):
    cid = lax.axis_index("c")
    sid = lax.axis_index("s")
    wid = sid * NC + cid
    r0 = sid * ROWS_PT

    pltpu.sync_copy(zrows.at[pl.ds(r0, ROWS_PT)], acc.at[pl.ds(r0, ROWS_PT)])
    plsc.subcore_barrier()

    @pl.loop(0, NH)
    def _(h):
        n = wid * NH + h
        pltpu.sync_copy(srcs.at[n], src_sl)
        pltpu.sync_copy(dsts.at[n], dst_sl)
        pltpu.async_copy(table.at[src_sl.at[0]], rows0, sem0)

        @pl.loop(0, K2, step=2)
        def _(j):
            pltpu.async_copy(table.at[src_sl.at[j + 1]], rows1, sem1)
            pltpu.make_async_copy(table.at[src_sl.at[j]], rows0, sem0).wait()
            pltpu.sync_copy(rows0, acc.at[dst_sl.at[j]], add=True)

            @pl.when(j + 2 < K2)
            def _():
                pltpu.async_copy(table.at[src_sl.at[j + 2]], rows0, sem0)

            pltpu.make_async_copy(table.at[src_sl.at[j + 1]], rows1,
                                  sem1).wait()
            pltpu.sync_copy(rows1, acc.at[dst_sl.at[j + 1]], add=True)

    plsc.subcore_barrier()
    pltpu.sync_copy(acc.at[pl.ds(r0, ROWS_PT)],
                    sums_out.at[cid, pl.ds(r0, ROWS_PT)])


_seg_sum = pl.kernel(
    _seg_sum_body,
    out_type=[jax.ShapeDtypeStruct((NC, NP, D), jnp.float32)],
    mesh=_mesh,
    scratch_types=[
        pltpu.VMEM_SHARED((NP, D), jnp.float32),
        pltpu.VMEM((K2, CH), jnp.int32),
        pltpu.VMEM((K2, CH), jnp.int32),
        pltpu.VMEM((CH, D), jnp.float32),
        pltpu.VMEM((CH, D), jnp.float32),
        pltpu.SemaphoreType.DMA,
        pltpu.SemaphoreType.DMA,
    ],
)


def _hist_body(dst_ref, out_ref):
    row = dst_ref[0]
    a = row // 128
    b = row - a * 128
    ii = lax.broadcasted_iota(jnp.int32, (128, EB), 0)
    a_oh = jnp.where(a == ii, 1.0, 0.0).astype(jnp.bfloat16)
    b_oh = jnp.where(b == ii, 1.0, 0.0).astype(jnp.bfloat16)
    part = lax.dot_general(a_oh, b_oh, (((1,), (1,)), ((), ())),
                           preferred_element_type=jnp.float32)

    @pl.when(pl.program_id(0) == 0)
    def _():
        out_ref[...] = jnp.zeros_like(out_ref)

    out_ref[...] += part


_hist = pl.pallas_call(
    _hist_body,
    grid=(HG,),
    in_specs=[pl.BlockSpec((1, 1, EB), lambda i: (i, 0, 0))],
    out_specs=pl.BlockSpec((128, 128), lambda i: (0, 0)),
    out_shape=jax.ShapeDtypeStruct((128, 128), jnp.float32),
)


def _tc1_body(sums, cnt, x, wl1, wr1, b1, wl2, wr2, b2, hw_out, hr_out):
    c = jnp.maximum(cnt[...], 1.0)
    mean = (sums[0, :N_NODES, :] + sums[1, :N_NODES, :]) / c
    h = mean @ wl1[...] + x[...] @ wr1[...] + b1[...][None, :]
    h = jnp.maximum(h, 0.0)
    hw_out[...] = h @ wl2[...]
    hr_out[...] = h @ wr2[...] + b2[...][None, :]


def _tc2_body(sums, cnt, hr, out):
    c = jnp.maximum(cnt[...], 1.0)
    out[...] = (sums[0, :N_NODES, :] + sums[1, :N_NODES, :]) / c + hr[...]


_tc1 = pl.pallas_call(
    _tc1_body,
    out_shape=(
        jax.ShapeDtypeStruct((N_NODES, D), jnp.float32),
        jax.ShapeDtypeStruct((N_NODES, D), jnp.float32),
    ),
)

_tc2 = pl.pallas_call(
    _tc2_body,
    out_shape=jax.ShapeDtypeStruct((N_NODES, D), jnp.float32),
)


def kernel(x, edge_index, Wl1, Wr1, b1, Wl2, Wr2, b2):
    src = edge_index[0].astype(jnp.int32)
    dst = edge_index[1].astype(jnp.int32)
    pad = E_PAD - N_EDGES
    ar = jnp.arange(pad, dtype=jnp.int32)
    srcs = jnp.concatenate(
        [src, ar % N_NODES]).reshape(NW * NH, K2, CH)
    dsts = jnp.concatenate(
        [dst, N_NODES + ar % (NP - N_NODES)]).reshape(NW * NH, K2, CH)

    zrows = jnp.zeros((NP, D), jnp.float32)

    cnt_grid = _hist(dst.reshape(HG, 1, EB))
    cnt = cnt_grid.reshape(-1)[:N_NODES, None]

    (sums1,) = _seg_sum(x, srcs, dsts, zrows)
    hw, hr = _tc1(sums1, cnt, x, Wl1, Wr1, b1, Wl2, Wr2, b2)
    (sums2,) = _seg_sum(hw, srcs, dsts, zrows)
    return _tc2(sums2, cnt, hr)
